# baseline (speedup 1.0000x reference)
import functools

import jax
import jax.numpy as jnp
from jax import lax
from jax.experimental import pallas as pl
from jax.experimental.pallas import tpu as pltpu
from jax.experimental.pallas import tpu_sc as plsc

N_ROWS = 4096
N_SUP = 256
N_QUERY = N_ROWS - N_SUP
D = 64
L = 16
DV = D // L
NC = 2
NS = 16
NW = NC * NS
QPW = N_QUERY // NW
SUPW = N_SUP // NS
SLOT = 128


@functools.partial(
    pl.kernel,
    mesh=plsc.VectorSubcoreMesh(core_axis_name="c", subcore_axis_name="s",
                                num_cores=NC),
    out_type=jax.ShapeDtypeStruct((NW, 2, L), jnp.float32),
    scratch_types=[
        pltpu.VMEM((SUPW, D), jnp.float32),
        pltpu.VMEM((SLOT,), jnp.float32),
        pltpu.VMEM_SHARED((NS * SLOT,), jnp.float32),
        pltpu.VMEM((NS * SLOT,), jnp.float32),
        pltpu.VMEM((QPW, D), jnp.float32),
        pltpu.VMEM((QPW,), jnp.int32),
        pltpu.VMEM((3 * L,), jnp.float32),
        pltpu.VMEM((3 * L,), jnp.float32),
        pltpu.VMEM((2, L), jnp.float32),
        pltpu.SemaphoreType.DMA,
        pltpu.SemaphoreType.DMA,
        pltpu.SemaphoreType.DMA,
    ],
)
def _proto_loss_sc(inp_hbm, tgt_hbm, out_hbm, sup_v, my_v, sh, all_v, q_v,
                   t_v, pad_d, pad_c, out_v, sem_s, sem_q, sem_t):
    c = lax.axis_index("c")
    s = lax.axis_index("s")
    w = s * NC + c
    qbase = N_SUP + w * QPW

    cp_s = pltpu.async_copy(inp_hbm.at[pl.ds(s * SUPW, SUPW)], sup_v, sem_s)
    cp_q = pltpu.async_copy(inp_hbm.at[pl.ds(qbase, QPW)], q_v, sem_q)
    cp_t = pltpu.async_copy(tgt_hbm.at[pl.ds(qbase, QPW)], t_v, sem_t)

    cp_s.wait()
    zeros16 = jnp.zeros((L,), jnp.float32)
    for j in range(DV):
        acc = sup_v[0, pl.ds(j * L, L)]
        for r in range(1, SUPW):
            acc = acc + sup_v[r, pl.ds(j * L, L)]
        my_v[pl.ds(j * L, L)] = acc
    for j in range(DV, SLOT // L):
        my_v[pl.ds(j * L, L)] = zeros16
    pltpu.sync_copy(my_v, sh.at[pl.ds(s * SLOT, SLOT)])

    cp_q.wait()
    QU = 12

    def qbody(r, carry):
        base = r * QU
        s1 = list(carry[:DV])
        s2 = list(carry[DV:])
        for i in range(QU):
            for j in range(DV):
                v = q_v[base + i, pl.ds(j * L, L)]
                s1[j] = s1[j] + v
                s2[j] = s2[j] + v * v
        return tuple(s1) + tuple(s2)

    mom = lax.fori_loop(0, QPW // QU, qbody,
                        tuple(zeros16 for _ in range(2 * DV)))
    s1 = mom[:DV]
    s2 = mom[DV:]

    cp_t.wait()
    cv = jnp.zeros((L,), jnp.float32)
    full_chunks = QPW // L
    for k in range(full_chunks):
        tc = t_v[pl.ds(k * L, L)]
        cv = cv + jnp.where(tc == 0, 1.0, 0.0).astype(jnp.float32)
    rem = QPW - full_chunks * L
    if rem:
        tc = t_v[pl.ds(QPW - L, L)]
        lane = lax.iota(jnp.int32, 16)
        m = (tc == 0) & (lane >= (L - rem))
        cv = cv + jnp.where(m, 1.0, 0.0).astype(jnp.float32)

    plsc.subcore_barrier()
    pltpu.sync_copy(sh, all_v)
    d2v = jnp.zeros((L,), jnp.float32)
    for j in range(DV):
        acc = all_v[pl.ds(j * L, L)]
        for r in range(1, NS):
            acc = acc + all_v[pl.ds(r * SLOT + j * L, L)]
        pj = acc * (1.0 / N_SUP)
        d2v = d2v + s2[j] - 2.0 * pj * s1[j] + float(QPW) * pj * pj

    pad_d[pl.ds(0, L)] = zeros16
    pad_d[pl.ds(2 * L, L)] = zeros16
    pad_c[pl.ds(0, L)] = zeros16
    pad_c[pl.ds(2 * L, L)] = zeros16
    for shift in (8, 4, 2, 1):
        pad_d[pl.ds(L, L)] = d2v
        pad_c[pl.ds(L, L)] = cv
        d2v = d2v + pad_d[pl.ds(L + shift, L)]
        cv = cv + pad_c[pl.ds(L + shift, L)]
    logit_sumv = -d2v
    lse_sumv = logit_sumv
    lossv = (lse_sumv - logit_sumv) * (1.0 / N_QUERY)
    accv = cv * (1.0 / N_QUERY)
    lane = lax.iota(jnp.int32, 16)
    m0 = lane == 0
    out_v[0, :] = jnp.where(m0, lossv, 0.0).astype(jnp.float32)
    out_v[1, :] = jnp.where(m0, accv, 0.0).astype(jnp.float32)
    pltpu.sync_copy(out_v, out_hbm.at[w])


def kernel(input, target):
    t32 = target.astype(jnp.int32)
    out = _proto_loss_sc(input, t32)
    loss = jnp.sum(out[:, 0, 0])
    acc = jnp.sum(out[:, 1, 0])
    return loss, acc


# speedup vs baseline: 1.0070x; 1.0070x over previous
import functools

import jax
import jax.numpy as jnp
from jax import lax
from jax.experimental import pallas as pl
from jax.experimental.pallas import tpu as pltpu
from jax.experimental.pallas import tpu_sc as plsc

N_ROWS = 4096
N_SUP = 256
N_QUERY = N_ROWS - N_SUP
D = 64
L = 16
DV = D // L
NC = 2
NS = 16
NW = NC * NS
QPW = N_QUERY // NW
SUPW = N_SUP // NS
SLOT = 128


@functools.partial(
    pl.kernel,
    mesh=plsc.VectorSubcoreMesh(core_axis_name="c", subcore_axis_name="s",
                                num_cores=NC),
    out_type=jax.ShapeDtypeStruct((NW, 2, L), jnp.float32),
    scratch_types=[
        pltpu.VMEM((SUPW, D), jnp.float32),
        pltpu.VMEM((SLOT,), jnp.float32),
        pltpu.VMEM_SHARED((NS * SLOT,), jnp.float32),
        pltpu.VMEM((NS * SLOT,), jnp.float32),
        pltpu.VMEM((QPW, D), jnp.float32),
        pltpu.VMEM((QPW,), jnp.int32),
        pltpu.VMEM((3 * L,), jnp.float32),
        pltpu.VMEM((3 * L,), jnp.float32),
        pltpu.VMEM((2, L), jnp.float32),
        pltpu.SemaphoreType.DMA,
        pltpu.SemaphoreType.DMA,
        pltpu.SemaphoreType.DMA,
    ],
)
def _proto_loss_sc(inp_hbm, tgt_hbm, out_hbm, sup_v, my_v, sh, all_v, q_v,
                   t_v, pad_d, pad_c, out_v, sem_s, sem_q, sem_t):
    c = lax.axis_index("c")
    s = lax.axis_index("s")
    w = s * NC + c
    qbase = N_SUP + w * QPW

    cp_s = pltpu.async_copy(inp_hbm.at[pl.ds(s * SUPW, SUPW)], sup_v, sem_s)
    cp_q = pltpu.async_copy(inp_hbm.at[pl.ds(qbase, QPW)], q_v, sem_q)
    cp_t = pltpu.async_copy(tgt_hbm.at[pl.ds(qbase, QPW)], t_v, sem_t)

    cp_s.wait()
    zeros16 = jnp.zeros((L,), jnp.float32)
    for j in range(DV):
        acc = sup_v[0, pl.ds(j * L, L)]
        for r in range(1, SUPW):
            acc = acc + sup_v[r, pl.ds(j * L, L)]
        my_v[pl.ds(j * L, L)] = acc
    for j in range(DV, SLOT // L):
        my_v[pl.ds(j * L, L)] = zeros16
    pltpu.sync_copy(my_v, sh.at[pl.ds(s * SLOT, SLOT)])
    plsc.subcore_barrier()
    pltpu.sync_copy(sh, all_v)
    proto = []
    for j in range(DV):
        acc = all_v[pl.ds(j * L, L)]
        for r in range(1, NS):
            acc = acc + all_v[pl.ds(r * SLOT + j * L, L)]
        proto.append(acc * (1.0 / N_SUP))

    cp_q.wait()
    QU = 12

    def qbody(r, carry):
        base = r * QU
        a0, a1 = carry
        for i in range(QU):
            sq = jnp.zeros((L,), jnp.float32)
            for j in range(DV):
                dvj = q_v[base + i, pl.ds(j * L, L)] - proto[j]
                sq = sq + dvj * dvj
            if i % 2 == 0:
                a0 = a0 + sq
            else:
                a1 = a1 + sq
        return (a0, a1)

    qa0, qa1 = lax.fori_loop(0, QPW // QU, qbody, (zeros16, zeros16))
    d2v = qa0 + qa1

    cp_t.wait()
    cv = jnp.zeros((L,), jnp.float32)
    full_chunks = QPW // L
    for k in range(full_chunks):
        tc = t_v[pl.ds(k * L, L)]
        cv = cv + jnp.where(tc == 0, 1.0, 0.0).astype(jnp.float32)
    rem = QPW - full_chunks * L
    if rem:
        tc = t_v[pl.ds(QPW - L, L)]
        lane = lax.iota(jnp.int32, 16)
        m = (tc == 0) & (lane >= (L - rem))
        cv = cv + jnp.where(m, 1.0, 0.0).astype(jnp.float32)

    pad_d[pl.ds(0, L)] = zeros16
    pad_d[pl.ds(2 * L, L)] = zeros16
    pad_c[pl.ds(0, L)] = zeros16
    pad_c[pl.ds(2 * L, L)] = zeros16
    for shift in (8, 4, 2, 1):
        pad_d[pl.ds(L, L)] = d2v
        pad_c[pl.ds(L, L)] = cv
        d2v = d2v + pad_d[pl.ds(L + shift, L)]
        cv = cv + pad_c[pl.ds(L + shift, L)]
    logit_sumv = -d2v
    lse_sumv = logit_sumv
    lossv = (lse_sumv - logit_sumv) * (1.0 / N_QUERY)
    accv = cv * (1.0 / N_QUERY)
    lane = lax.iota(jnp.int32, 16)
    m0 = lane == 0
    out_v[0, :] = jnp.where(m0, lossv, 0.0).astype(jnp.float32)
    out_v[1, :] = jnp.where(m0, accv, 0.0).astype(jnp.float32)
    pltpu.sync_copy(out_v, out_hbm.at[w])


def kernel(input, target):
    t32 = target.astype(jnp.int32)
    out = _proto_loss_sc(input, t32)
    loss = jnp.sum(out[:, 0, 0])
    acc = jnp.sum(out[:, 1, 0])
    return loss, acc
